# baseline (device time: 29505 ns/iter reference)
import jax
import jax.numpy as jnp
from jax import lax
from jax.experimental import pallas as pl
from jax.experimental.pallas import tpu as pltpu

N_DEV = 8
B = 2
SQ = 256
HALO = 128
KV_BUF = SQ + 2 * HALO
HQ = 4
DH = 64
SKV_GLOBAL = N_DEV * SQ


def kernel(x, Wq, K_ext, V_ext, Wo):
    def body(x_ref, wq_ref, k_ref, v_ref, wo_ref, out_ref,
             kbuf, vbuf, send_sems, recv_sems):
        my = lax.axis_index("i")
        left = jnp.maximum(my - 1, 0)
        right = jnp.minimum(my + 1, N_DEV - 1)

        kbuf[:, HALO:HALO + SQ] = k_ref[...]
        vbuf[:, HALO:HALO + SQ] = v_ref[...]

        @pl.when(my == 0)
        def _():
            kbuf[:, 0:HALO] = jnp.zeros((B, HALO, HQ, DH), jnp.float32)
            vbuf[:, 0:HALO] = jnp.zeros((B, HALO, HQ, DH), jnp.float32)

        @pl.when(my == N_DEV - 1)
        def _():
            kbuf[:, HALO + SQ:] = jnp.zeros((B, HALO, HQ, DH), jnp.float32)
            vbuf[:, HALO + SQ:] = jnp.zeros((B, HALO, HQ, DH), jnp.float32)

        rdma_r_k = pltpu.make_async_remote_copy(
            src_ref=k_ref.at[:, pl.ds(SQ - HALO, HALO)],
            dst_ref=kbuf.at[:, pl.ds(0, HALO)],
            send_sem=send_sems.at[0], recv_sem=recv_sems.at[0],
            device_id=(right,), device_id_type=pltpu.DeviceIdType.MESH,
        )
        rdma_r_v = pltpu.make_async_remote_copy(
            src_ref=v_ref.at[:, pl.ds(SQ - HALO, HALO)],
            dst_ref=vbuf.at[:, pl.ds(0, HALO)],
            send_sem=send_sems.at[1], recv_sem=recv_sems.at[1],
            device_id=(right,), device_id_type=pltpu.DeviceIdType.MESH,
        )
        rdma_l_k = pltpu.make_async_remote_copy(
            src_ref=k_ref.at[:, pl.ds(0, HALO)],
            dst_ref=kbuf.at[:, pl.ds(HALO + SQ, HALO)],
            send_sem=send_sems.at[2], recv_sem=recv_sems.at[2],
            device_id=(left,), device_id_type=pltpu.DeviceIdType.MESH,
        )
        rdma_l_v = pltpu.make_async_remote_copy(
            src_ref=v_ref.at[:, pl.ds(0, HALO)],
            dst_ref=vbuf.at[:, pl.ds(HALO + SQ, HALO)],
            send_sem=send_sems.at[3], recv_sem=recv_sems.at[3],
            device_id=(left,), device_id_type=pltpu.DeviceIdType.MESH,
        )

        @pl.when(my < N_DEV - 1)
        def _():
            rdma_r_k.start()
            rdma_r_v.start()

        @pl.when(my > 0)
        def _():
            rdma_l_k.start()
            rdma_l_v.start()

        q = [
            jnp.dot(x_ref[b], wq_ref[...],
                    preferred_element_type=jnp.float32)
            for b in range(B)
        ]

        @pl.when(my > 0)
        def _():
            rdma_r_k.wait_recv()
            rdma_r_v.wait_recv()

        @pl.when(my < N_DEV - 1)
        def _():
            rdma_l_k.wait_recv()
            rdma_l_v.wait_recv()

        @pl.when(my < N_DEV - 1)
        def _():
            rdma_r_k.wait_send()
            rdma_r_v.wait_send()

        @pl.when(my > 0)
        def _():
            rdma_l_k.wait_send()
            rdma_l_v.wait_send()

        r_idx = lax.broadcasted_iota(jnp.int32, (SQ, KV_BUF), 0)
        j_idx = lax.broadcasted_iota(jnp.int32, (SQ, KV_BUF), 1)
        d = j_idx - r_idx
        kglob = my * SQ - HALO + j_idx
        mask = (d >= 0) & (d <= 2 * HALO) & (kglob >= 0) & (kglob < SKV_GLOBAL)

        for b in range(B):
            acc = jnp.zeros((SQ, x_ref.shape[2]), jnp.float32)
            for h in range(HQ):
                qh = q[b][:, h * DH:(h + 1) * DH]
                kh = kbuf[b, :, h, :]
                vh = vbuf[b, :, h, :]
                s = lax.dot_general(
                    qh, kh, (((1,), (1,)), ((), ())),
                    preferred_element_type=jnp.float32,
                ) * 0.125
                s = jnp.where(mask, s, -1e9)
                m = jnp.max(s, axis=1, keepdims=True)
                w = jnp.exp(s - m)
                w = w / jnp.sum(w, axis=1, keepdims=True)
                ctx_h = jnp.dot(w, vh, preferred_element_type=jnp.float32)
                acc += jnp.dot(ctx_h, wo_ref[h * DH:(h + 1) * DH, :],
                               preferred_element_type=jnp.float32)
            out_ref[b] = acc

    return pl.pallas_call(
        body,
        out_shape=jax.ShapeDtypeStruct(x.shape, jnp.float32),
        in_specs=[pl.BlockSpec(memory_space=pltpu.VMEM)] * 5,
        out_specs=pl.BlockSpec(memory_space=pltpu.VMEM),
        scratch_shapes=[
            pltpu.VMEM((B, KV_BUF, HQ, DH), jnp.float32),
            pltpu.VMEM((B, KV_BUF, HQ, DH), jnp.float32),
            pltpu.SemaphoreType.DMA((4,)),
            pltpu.SemaphoreType.DMA((4,)),
        ],
    )(x, Wq, K_ext, V_ext, Wo)
